# baseline (speedup 1.0000x reference)
import jax
import jax.numpy as jnp
from jax import lax
from jax.experimental import pallas as pl
from jax.experimental.pallas import tpu as pltpu
from jax.experimental.pallas import tpu_sc as plsc

N = 10000
E = 320000
D_IN = 128
D_HID = 256
D_OUT = 128
N_CLASSES = 64

NC = 2
NS = 16
LANES = 16

EB = 100
IBC1 = 20
NIC1 = E // NS // EB // IBC1
IBC2 = 20
NIC2 = E // (NC * NS) // EB // IBC2
IDEG = 40
NDEG = E // NS // EB // IDEG

RCHUNK = 16
CPT = 39
NTAIL = N - CPT * NS * RCHUNK

MB = 1000


def _sc_mesh():
    return plsc.VectorSubcoreMesh(core_axis_name="c", subcore_axis_name="s")


def _sc_degrees(e5):
    nbt = E // NS // EB

    def body(e5_ref, dsrc_ref, ddst_ref, idx_v, ones_v, zbuf_v, acc_sp,
             semD):
        cid = lax.axis_index("c")
        sid = lax.axis_index("s")
        for t in range(112 // LANES):
            ones_v[pl.ds(t * LANES, LANES)] = jnp.ones((LANES,), jnp.float32)

        def zstep(k, _):
            zbuf_v[pl.ds(k * LANES, LANES)] = jnp.zeros((LANES,), jnp.float32)
            return 0

        lax.fori_loop(0, N // LANES, zstep, 0)

        @pl.when(sid == 0)
        def _():
            pltpu.sync_copy(zbuf_v, acc_sp)

        plsc.subcore_barrier()

        def chunk(ci, _):
            pltpu.sync_copy(e5_ref.at[cid, sid, ci], idx_v)

            def group(g, _):
                def fire(j, _):
                    pltpu.async_copy(ones_v.at[pl.ds(0, EB)],
                                     acc_sp.at[idx_v.at[g * 8 + j]],
                                     semD, add=True)
                    return 0

                def drain(j, _):
                    pltpu.make_async_copy(
                        ones_v.at[pl.ds(0, EB)],
                        acc_sp.at[idx_v.at[g * 8 + j]], semD).wait()
                    return 0

                lax.fori_loop(0, 8, fire, 0)
                lax.fori_loop(0, 8, drain, 0)
                return 0

            lax.fori_loop(0, IDEG // 8, group, 0)
            return 0

        lax.fori_loop(0, NDEG, chunk, 0)
        plsc.subcore_barrier()

        @pl.when(sid == 0)
        def _():
            pltpu.sync_copy(acc_sp, zbuf_v)

            @pl.when(cid == 0)
            def _():
                pltpu.sync_copy(zbuf_v, dsrc_ref)

            @pl.when(cid == 1)
            def _():
                pltpu.sync_copy(zbuf_v, ddst_ref)

    del nbt
    f = pl.kernel(
        body,
        out_type=(
            jax.ShapeDtypeStruct((N,), jnp.float32),
            jax.ShapeDtypeStruct((N,), jnp.float32),
        ),
        mesh=_sc_mesh(),
        scratch_types=[
            pltpu.VMEM((IDEG, EB), jnp.int32),
            pltpu.VMEM((112,), jnp.float32),
            pltpu.VMEM((N,), jnp.float32),
            pltpu.VMEM_SHARED((N,), jnp.float32),
            pltpu.SemaphoreType.DMA,
        ],
    )
    return f(e5)


def _zero_acc(buf_v, acc_sp, sid, dc):
    def zstep(r, _):
        for t in range(dc // LANES):
            buf_v[r, pl.ds(t * LANES, LANES)] = jnp.zeros(
                (LANES,), jnp.float32)
        return 0

    lax.fori_loop(0, RCHUNK, zstep, 0)
    for t in range(CPT):
        base = (t * NS + sid) * RCHUNK
        pltpu.sync_copy(buf_v, acc_sp.at[pl.ds(base, RCHUNK)])

    @pl.when(sid == 0)
    def _():
        pltpu.sync_copy(buf_v.at[pl.ds(0, NTAIL)],
                        acc_sp.at[pl.ds(CPT * NS * RCHUNK, NTAIL)])


def _copy_out_acc(buf_v, acc_sp, agg_ref, cid, sid):
    for t in range(CPT):
        base = (t * NS + sid) * RCHUNK
        pltpu.sync_copy(acc_sp.at[pl.ds(base, RCHUNK)], buf_v)
        pltpu.sync_copy(buf_v, agg_ref.at[cid, pl.ds(base, RCHUNK)])

    @pl.when(sid == 0)
    def _():
        tbase = CPT * NS * RCHUNK
        pltpu.sync_copy(acc_sp.at[pl.ds(tbase, NTAIL)],
                        buf_v.at[pl.ds(0, NTAIL)])
        pltpu.sync_copy(buf_v.at[pl.ds(0, NTAIL)],
                        agg_ref.at[cid, pl.ds(tbase, NTAIL)])


def _pipelined_edge_pass(issue_gather, wait_gather, didx_v, bufs,
                         acc_sp, semS, n):
    def issue_scatter(j):
        pltpu.async_copy(bufs[j % 3], acc_sp.at[didx_v.at[j]],
                         semS, add=True)

    def wait_scatter(j):
        pltpu.make_async_copy(bufs[j % 3], acc_sp.at[didx_v.at[j]],
                              semS).wait()

    issue_gather(0, bufs[0])
    issue_gather(1, bufs[1])
    for j in range(n):
        if j >= 1:
            wait_scatter(j - 1)
        if j + 2 < n:
            issue_gather(j + 2, bufs[(j + 2) % 3])
        wait_gather(j, bufs[j % 3])
        issue_scatter(j)
    wait_scatter(n - 1)


def _sc_gs_edgesplit(hs, e5):
    def body(hs_ref, e5_ref, agg_ref,
             sidx_v, didx_v, r0_v, r1_v, r2_v, buf_v, acc_sp, semG, semS):
        cid = lax.axis_index("c")
        sid = lax.axis_index("s")
        wid = cid * NS + sid
        _zero_acc(buf_v, acc_sp, sid, D_OUT)
        plsc.subcore_barrier()

        def issue_gather(j, buf):
            pltpu.async_copy(hs_ref.at[sidx_v.at[j]], buf, semG)

        def wait_gather(j, buf):
            pltpu.make_async_copy(hs_ref.at[sidx_v.at[j]], buf, semG).wait()

        def chunk(ci, _):
            pltpu.sync_copy(e5_ref.at[0, wid, ci], sidx_v)
            pltpu.sync_copy(e5_ref.at[1, wid, ci], didx_v)
            _pipelined_edge_pass(issue_gather, wait_gather, didx_v,
                                 (r0_v, r1_v, r2_v), acc_sp, semS, IBC2)
            return 0

        lax.fori_loop(0, NIC2, chunk, 0)
        plsc.subcore_barrier()
        _copy_out_acc(buf_v, acc_sp, agg_ref, cid, sid)

    f = pl.kernel(
        body,
        out_type=jax.ShapeDtypeStruct((NC, N, D_OUT), jnp.float32),
        mesh=_sc_mesh(),
        scratch_types=[
            pltpu.VMEM((IBC2, EB), jnp.int32),
            pltpu.VMEM((IBC2, EB), jnp.int32),
            pltpu.VMEM((EB, D_OUT), jnp.float32),
            pltpu.VMEM((EB, D_OUT), jnp.float32),
            pltpu.VMEM((EB, D_OUT), jnp.float32),
            pltpu.VMEM((RCHUNK, D_OUT), jnp.float32),
            pltpu.VMEM_SHARED((N, D_OUT), jnp.float32),
            pltpu.SemaphoreType.DMA,
            pltpu.SemaphoreType.DMA,
        ],
    )
    return f(hs, e5)


def _dinv(deg):
    return jnp.where(deg > 0.0,
                     lax.rsqrt(jnp.maximum(deg, 1.0)),
                     0.0)


def _tc_pre(x, deg_src, deg_dst):
    def body(x_ref, ds_ref, dd_ref, xs_ref, dis_ref, did_ref):
        dis = _dinv(ds_ref[...])
        did = _dinv(dd_ref[...])
        xs_ref[...] = x_ref[...] * dis
        dis_ref[...] = dis
        did_ref[...] = did

    return pl.pallas_call(
        body,
        grid=(N // MB,),
        in_specs=[
            pl.BlockSpec((MB, D_IN), lambda i: (i, 0)),
            pl.BlockSpec((MB, 1), lambda i: (i, 0)),
            pl.BlockSpec((MB, 1), lambda i: (i, 0)),
        ],
        out_specs=[
            pl.BlockSpec((MB, D_IN), lambda i: (i, 0)),
            pl.BlockSpec((MB, 1), lambda i: (i, 0)),
            pl.BlockSpec((MB, 1), lambda i: (i, 0)),
        ],
        out_shape=[
            jax.ShapeDtypeStruct((N, D_IN), jnp.float32),
            jax.ShapeDtypeStruct((N, 1), jnp.float32),
            jax.ShapeDtypeStruct((N, 1), jnp.float32),
        ],
    )(x, deg_src, deg_dst)


def _tc_mid1(p1, dinv_dst, dinv_src, b1, W1, W2):
    def body(a_ref, dd_ref, ds_ref, b1_ref, w1_ref, w2_ref, o_ref):
        aggx = a_ref[0] + a_ref[1]
        aggx = aggx * dd_ref[...]
        h = jnp.dot(aggx, w1_ref[...], preferred_element_type=jnp.float32)
        h = jnp.maximum(h + b1_ref[...][None, :], 0.0)
        hs2 = jnp.dot(h, w2_ref[...], preferred_element_type=jnp.float32)
        o_ref[...] = hs2 * ds_ref[...]

    return pl.pallas_call(
        body,
        grid=(N // MB,),
        in_specs=[
            pl.BlockSpec((NC, MB, D_IN), lambda i: (0, i, 0)),
            pl.BlockSpec((MB, 1), lambda i: (i, 0)),
            pl.BlockSpec((MB, 1), lambda i: (i, 0)),
            pl.BlockSpec((D_HID,), lambda i: (0,)),
            pl.BlockSpec((D_IN, D_HID), lambda i: (0, 0)),
            pl.BlockSpec((D_HID, D_OUT), lambda i: (0, 0)),
        ],
        out_specs=pl.BlockSpec((MB, D_OUT), lambda i: (i, 0)),
        out_shape=jax.ShapeDtypeStruct((N, D_OUT), jnp.float32),
    )(p1, dinv_dst, dinv_src, b1, W1, W2)


def _tc_head(agg2, dinv_dst, b2, Wh, bh):
    def body(a_ref, dd_ref, b2_ref, wh_ref, bh_ref, o_ref):
        h = a_ref[0] + a_ref[1]
        h = h * dd_ref[...] + b2_ref[...][None, :]
        o_ref[...] = (jnp.dot(h, wh_ref[...],
                              preferred_element_type=jnp.float32)
                      + bh_ref[...][None, :])

    return pl.pallas_call(
        body,
        grid=(N // MB,),
        in_specs=[
            pl.BlockSpec((NC, MB, D_OUT), lambda i: (0, i, 0)),
            pl.BlockSpec((MB, 1), lambda i: (i, 0)),
            pl.BlockSpec((D_OUT,), lambda i: (0,)),
            pl.BlockSpec((D_OUT, N_CLASSES), lambda i: (0, 0)),
            pl.BlockSpec((N_CLASSES,), lambda i: (0,)),
        ],
        out_specs=pl.BlockSpec((MB, N_CLASSES), lambda i: (i, 0)),
        out_shape=jax.ShapeDtypeStruct((N, N_CLASSES), jnp.float32),
    )(agg2, dinv_dst, b2, Wh, bh)


def kernel(x, edge_index, i, W1, b1, W2, b2, Wh, bh):
    del i
    e5b = edge_index.reshape(2, NC * NS, NIC2, IBC2, EB)
    e5d = edge_index.reshape(2, NS, NDEG, IDEG, EB)
    deg_src, deg_dst = _sc_degrees(e5d)
    xs, dinv_src, dinv_dst = _tc_pre(
        x, deg_src.reshape(N, 1), deg_dst.reshape(N, 1))
    p1 = _sc_gs_edgesplit(xs, e5b)
    hs2 = _tc_mid1(p1, dinv_dst, dinv_src, b1, W1, W2)
    p2 = _sc_gs_edgesplit(hs2, e5b)
    return _tc_head(p2, dinv_dst, b2, Wh, bh)


# speedup vs baseline: 1.0398x; 1.0398x over previous
import jax
import jax.numpy as jnp
from jax import lax
from jax.experimental import pallas as pl
from jax.experimental.pallas import tpu as pltpu
from jax.experimental.pallas import tpu_sc as plsc

N = 10000
E = 320000
D_IN = 128
D_HID = 256
D_OUT = 128
N_CLASSES = 64

NC = 2
NS = 16
LANES = 16

EB = 100
IBC1 = 20
NIC1 = E // NS // EB // IBC1
EB2 = 80
IBC2 = 25
NIC2 = E // (NC * NS) // EB2 // IBC2
IDEG = 40
NDEG = E // NS // EB // IDEG

RCHUNK = 16
CPT = 39
NTAIL = N - CPT * NS * RCHUNK

MB = 1000


def _sc_mesh():
    return plsc.VectorSubcoreMesh(core_axis_name="c", subcore_axis_name="s")


def _sc_degrees(e5):
    nbt = E // NS // EB

    def body(e5_ref, dsrc_ref, ddst_ref, idx_v, ones_v, zbuf_v, acc_sp,
             semD):
        cid = lax.axis_index("c")
        sid = lax.axis_index("s")
        for t in range(112 // LANES):
            ones_v[pl.ds(t * LANES, LANES)] = jnp.ones((LANES,), jnp.float32)

        def zstep(k, _):
            zbuf_v[pl.ds(k * LANES, LANES)] = jnp.zeros((LANES,), jnp.float32)
            return 0

        lax.fori_loop(0, N // LANES, zstep, 0)

        @pl.when(sid == 0)
        def _():
            pltpu.sync_copy(zbuf_v, acc_sp)

        plsc.subcore_barrier()

        def chunk(ci, _):
            pltpu.sync_copy(e5_ref.at[cid, sid, ci], idx_v)

            def group(g, _):
                def fire(j, _):
                    pltpu.async_copy(ones_v.at[pl.ds(0, EB)],
                                     acc_sp.at[idx_v.at[g * 8 + j]],
                                     semD, add=True)
                    return 0

                def drain(j, _):
                    pltpu.make_async_copy(
                        ones_v.at[pl.ds(0, EB)],
                        acc_sp.at[idx_v.at[g * 8 + j]], semD).wait()
                    return 0

                lax.fori_loop(0, 8, fire, 0)
                lax.fori_loop(0, 8, drain, 0)
                return 0

            lax.fori_loop(0, IDEG // 8, group, 0)
            return 0

        lax.fori_loop(0, NDEG, chunk, 0)
        plsc.subcore_barrier()

        @pl.when(sid == 0)
        def _():
            pltpu.sync_copy(acc_sp, zbuf_v)

            @pl.when(cid == 0)
            def _():
                pltpu.sync_copy(zbuf_v, dsrc_ref)

            @pl.when(cid == 1)
            def _():
                pltpu.sync_copy(zbuf_v, ddst_ref)

    del nbt
    f = pl.kernel(
        body,
        out_type=(
            jax.ShapeDtypeStruct((N,), jnp.float32),
            jax.ShapeDtypeStruct((N,), jnp.float32),
        ),
        mesh=_sc_mesh(),
        scratch_types=[
            pltpu.VMEM((IDEG, EB), jnp.int32),
            pltpu.VMEM((112,), jnp.float32),
            pltpu.VMEM((N,), jnp.float32),
            pltpu.VMEM_SHARED((N,), jnp.float32),
            pltpu.SemaphoreType.DMA,
        ],
    )
    return f(e5)


def _zero_acc(buf_v, acc_sp, sid, dc):
    def zstep(r, _):
        for t in range(dc // LANES):
            buf_v[r, pl.ds(t * LANES, LANES)] = jnp.zeros(
                (LANES,), jnp.float32)
        return 0

    lax.fori_loop(0, RCHUNK, zstep, 0)
    for t in range(CPT):
        base = (t * NS + sid) * RCHUNK
        pltpu.sync_copy(buf_v, acc_sp.at[pl.ds(base, RCHUNK)])

    @pl.when(sid == 0)
    def _():
        pltpu.sync_copy(buf_v.at[pl.ds(0, NTAIL)],
                        acc_sp.at[pl.ds(CPT * NS * RCHUNK, NTAIL)])


def _copy_out_acc(buf_v, acc_sp, agg_ref, cid, sid):
    for t in range(CPT):
        base = (t * NS + sid) * RCHUNK
        pltpu.sync_copy(acc_sp.at[pl.ds(base, RCHUNK)], buf_v)
        pltpu.sync_copy(buf_v, agg_ref.at[cid, pl.ds(base, RCHUNK)])

    @pl.when(sid == 0)
    def _():
        tbase = CPT * NS * RCHUNK
        pltpu.sync_copy(acc_sp.at[pl.ds(tbase, NTAIL)],
                        buf_v.at[pl.ds(0, NTAIL)])
        pltpu.sync_copy(buf_v.at[pl.ds(0, NTAIL)],
                        agg_ref.at[cid, pl.ds(tbase, NTAIL)])


def _sc_gs_edgesplit(hs, e5):
    n = NIC2 * IBC2

    def body(hs_ref, e5_ref, agg_ref, sA0, dA0, sA1, dA1,
             r0, r1, r2, buf_v, acc_sp, semG, semS, semI):
        cid = lax.axis_index("c")
        sid = lax.axis_index("s")
        wid = cid * NS + sid
        _zero_acc(buf_v, acc_sp, sid, D_OUT)
        plsc.subcore_barrier()

        spair = (sA0, sA1)
        dpair = (dA0, dA1)
        bufs = (r0, r1, r2)
        pltpu.sync_copy(e5_ref.at[0, wid, 0], sA0)
        pltpu.sync_copy(e5_ref.at[1, wid, 0], dA0)

        def sref(g):
            return spair[(g // IBC2) % 2].at[g % IBC2]

        def dref(g):
            return dpair[(g // IBC2) % 2].at[g % IBC2]

        def issue_gather(g):
            pltpu.async_copy(hs_ref.at[sref(g)], bufs[g % 3], semG)

        def wait_gather(g):
            pltpu.make_async_copy(hs_ref.at[sref(g)], bufs[g % 3],
                                  semG).wait()

        def issue_scatter(g):
            pltpu.async_copy(bufs[g % 3], acc_sp.at[dref(g)],
                             semS, add=True)

        def wait_scatter(g):
            pltpu.make_async_copy(bufs[g % 3], acc_sp.at[dref(g)],
                                  semS).wait()

        def issue_idx(c):
            pltpu.async_copy(e5_ref.at[0, wid, c], spair[c % 2], semI)
            pltpu.async_copy(e5_ref.at[1, wid, c], dpair[c % 2], semI)

        def wait_idx(c):
            pltpu.make_async_copy(e5_ref.at[0, wid, c], spair[c % 2],
                                  semI).wait()
            pltpu.make_async_copy(e5_ref.at[1, wid, c], dpair[c % 2],
                                  semI).wait()

        issue_gather(0)
        issue_gather(1)
        for g in range(n):
            c = g // IBC2
            if g >= 1:
                wait_scatter(g - 1)
            if g % IBC2 == 0 and c + 1 < NIC2:
                issue_idx(c + 1)
            if g + 2 < n:
                if (g + 2) % IBC2 == 0:
                    wait_idx((g + 2) // IBC2)
                issue_gather(g + 2)
            wait_gather(g)
            issue_scatter(g)
        wait_scatter(n - 1)

        plsc.subcore_barrier()
        _copy_out_acc(buf_v, acc_sp, agg_ref, cid, sid)

    f = pl.kernel(
        body,
        out_type=jax.ShapeDtypeStruct((NC, N, D_OUT), jnp.float32),
        mesh=_sc_mesh(),
        scratch_types=[
            pltpu.VMEM((IBC2, EB2), jnp.int32),
            pltpu.VMEM((IBC2, EB2), jnp.int32),
            pltpu.VMEM((IBC2, EB2), jnp.int32),
            pltpu.VMEM((IBC2, EB2), jnp.int32),
            pltpu.VMEM((EB2, D_OUT), jnp.float32),
            pltpu.VMEM((EB2, D_OUT), jnp.float32),
            pltpu.VMEM((EB2, D_OUT), jnp.float32),
            pltpu.VMEM((RCHUNK, D_OUT), jnp.float32),
            pltpu.VMEM_SHARED((N, D_OUT), jnp.float32),
            pltpu.SemaphoreType.DMA,
            pltpu.SemaphoreType.DMA,
            pltpu.SemaphoreType.DMA,
        ],
    )
    return f(hs, e5)


def _dinv(deg):
    return jnp.where(deg > 0.0,
                     lax.rsqrt(jnp.maximum(deg, 1.0)),
                     0.0)


def _tc_pre(x, deg_src, deg_dst):
    def body(x_ref, ds_ref, dd_ref, xs_ref, dis_ref, did_ref):
        dis = _dinv(ds_ref[...])
        did = _dinv(dd_ref[...])
        xs_ref[...] = x_ref[...] * dis
        dis_ref[...] = dis
        did_ref[...] = did

    return pl.pallas_call(
        body,
        grid=(N // MB,),
        in_specs=[
            pl.BlockSpec((MB, D_IN), lambda i: (i, 0)),
            pl.BlockSpec((MB, 1), lambda i: (i, 0)),
            pl.BlockSpec((MB, 1), lambda i: (i, 0)),
        ],
        out_specs=[
            pl.BlockSpec((MB, D_IN), lambda i: (i, 0)),
            pl.BlockSpec((MB, 1), lambda i: (i, 0)),
            pl.BlockSpec((MB, 1), lambda i: (i, 0)),
        ],
        out_shape=[
            jax.ShapeDtypeStruct((N, D_IN), jnp.float32),
            jax.ShapeDtypeStruct((N, 1), jnp.float32),
            jax.ShapeDtypeStruct((N, 1), jnp.float32),
        ],
    )(x, deg_src, deg_dst)


def _tc_mid1(p1, dinv_dst, dinv_src, b1, W1, W2):
    def body(a_ref, dd_ref, ds_ref, b1_ref, w1_ref, w2_ref, o_ref):
        aggx = a_ref[0] + a_ref[1]
        aggx = aggx * dd_ref[...]
        h = jnp.dot(aggx, w1_ref[...], preferred_element_type=jnp.float32)
        h = jnp.maximum(h + b1_ref[...][None, :], 0.0)
        hs2 = jnp.dot(h, w2_ref[...], preferred_element_type=jnp.float32)
        o_ref[...] = hs2 * ds_ref[...]

    return pl.pallas_call(
        body,
        grid=(N // MB,),
        in_specs=[
            pl.BlockSpec((NC, MB, D_IN), lambda i: (0, i, 0)),
            pl.BlockSpec((MB, 1), lambda i: (i, 0)),
            pl.BlockSpec((MB, 1), lambda i: (i, 0)),
            pl.BlockSpec((D_HID,), lambda i: (0,)),
            pl.BlockSpec((D_IN, D_HID), lambda i: (0, 0)),
            pl.BlockSpec((D_HID, D_OUT), lambda i: (0, 0)),
        ],
        out_specs=pl.BlockSpec((MB, D_OUT), lambda i: (i, 0)),
        out_shape=jax.ShapeDtypeStruct((N, D_OUT), jnp.float32),
    )(p1, dinv_dst, dinv_src, b1, W1, W2)


def _tc_head(agg2, dinv_dst, b2, Wh, bh):
    def body(a_ref, dd_ref, b2_ref, wh_ref, bh_ref, o_ref):
        h = a_ref[0] + a_ref[1]
        h = h * dd_ref[...] + b2_ref[...][None, :]
        o_ref[...] = (jnp.dot(h, wh_ref[...],
                              preferred_element_type=jnp.float32)
                      + bh_ref[...][None, :])

    return pl.pallas_call(
        body,
        grid=(N // MB,),
        in_specs=[
            pl.BlockSpec((NC, MB, D_OUT), lambda i: (0, i, 0)),
            pl.BlockSpec((MB, 1), lambda i: (i, 0)),
            pl.BlockSpec((D_OUT,), lambda i: (0,)),
            pl.BlockSpec((D_OUT, N_CLASSES), lambda i: (0, 0)),
            pl.BlockSpec((N_CLASSES,), lambda i: (0,)),
        ],
        out_specs=pl.BlockSpec((MB, N_CLASSES), lambda i: (i, 0)),
        out_shape=jax.ShapeDtypeStruct((N, N_CLASSES), jnp.float32),
    )(agg2, dinv_dst, b2, Wh, bh)


def kernel(x, edge_index, i, W1, b1, W2, b2, Wh, bh):
    del i
    e5b = edge_index.reshape(2, NC * NS, NIC2, IBC2, EB2)
    e5d = edge_index.reshape(2, NS, NDEG, IDEG, EB)
    deg_src, deg_dst = _sc_degrees(e5d)
    xs, dinv_src, dinv_dst = _tc_pre(
        x, deg_src.reshape(N, 1), deg_dst.reshape(N, 1))
    p1 = _sc_gs_edgesplit(xs, e5b)
    hs2 = _tc_mid1(p1, dinv_dst, dinv_src, b1, W1, W2)
    p2 = _sc_gs_edgesplit(hs2, e5b)
    return _tc_head(p2, dinv_dst, b2, Wh, bh)
